# Initial kernel scaffold; baseline (speedup 1.0000x reference)
import jax
import jax.numpy as jnp
from jax.experimental import pallas as pl


def kernel(QR, symbols, W, b):
    raise NotImplementedError("write your pallas kernel here")


# speedup vs baseline: 2.0782x; 2.0782x over previous
import functools

import jax
import jax.numpy as jnp
from jax import lax
from jax.experimental import pallas as pl
from jax.experimental.pallas import tpu as pltpu
from jax.experimental.pallas import tpu_sc as plsc

V = 63
FEAT = 256
D = 64
B = 4096
L = 200
BT = B * L

NC = 2
NS = 16
NW = NC * NS
CH = 128
PER_W = BT // NW
N_CHUNKS = PER_W // CH


def _table_body(sym_ref, w_ref, b_ref, out_ref):
    out_ref[...] = (
        jnp.dot(sym_ref[...], w_ref[...], preferred_element_type=jnp.float32)
        + b_ref[...]
    )


def _make_table(symbols, W, b):
    sym_pad = jnp.pad(symbols, ((0, 64 - V), (0, 0)))
    return pl.pallas_call(
        _table_body,
        out_shape=jax.ShapeDtypeStruct((64, D), jnp.float32),
    )(sym_pad, W, b.reshape(1, D))


def _sc_gather_body(table_hbm, idx_hbm, out_hbm, idx_v, rows_v, sem):
    wid = lax.axis_index("s") * NC + lax.axis_index("c")
    pltpu.sync_copy(idx_hbm.at[pl.ds(wid * N_CHUNKS, N_CHUNKS)], idx_v)

    def body(ci, _):
        pltpu.async_copy(table_hbm.at[idx_v.at[ci]], rows_v, sem).wait()
        pltpu.sync_copy(
            rows_v, out_hbm.at[pl.ds((wid * N_CHUNKS + ci) * CH, CH)]
        )
        return _

    lax.fori_loop(0, N_CHUNKS, body, None)


@functools.partial(jax.jit)
def kernel(QR, symbols, W, b):
    table = _make_table(symbols, W, b)
    idx = QR.reshape(NW * N_CHUNKS, CH).astype(jnp.int32)
    mesh = plsc.VectorSubcoreMesh(core_axis_name="c", subcore_axis_name="s")
    gather = pl.kernel(
        _sc_gather_body,
        out_type=jax.ShapeDtypeStruct((BT, D), jnp.float32),
        mesh=mesh,
        scratch_types=[
            pltpu.VMEM((N_CHUNKS, CH), jnp.int32),
            pltpu.VMEM((CH, D), jnp.float32),
            pltpu.SemaphoreType.DMA,
        ],
        compiler_params=pltpu.CompilerParams(use_tc_tiling_on_sc=False),
    )
    out = gather(table, idx)
    return out.reshape(B, L, D)


# speedup vs baseline: 2.0887x; 1.0050x over previous
import functools

import jax
import jax.numpy as jnp
from jax import lax
from jax.experimental import pallas as pl
from jax.experimental.pallas import tpu as pltpu
from jax.experimental.pallas import tpu_sc as plsc

V = 63
FEAT = 256
D = 64
B = 4096
L = 200
BT = B * L

NC = 2
NS = 16
NW = NC * NS
CH = 128
PER_W = BT // NW
N_CHUNKS = PER_W // CH
NBUF = 4


def _table_body(sym_ref, w_ref, b_ref, out_ref):
    out_ref[...] = (
        jnp.dot(sym_ref[...], w_ref[...], preferred_element_type=jnp.float32)
        + b_ref[...]
    )


def _make_table(symbols, W, b):
    sym_pad = jnp.pad(symbols, ((0, 64 - V), (0, 0)))
    return pl.pallas_call(
        _table_body,
        out_shape=jax.ShapeDtypeStruct((64, D), jnp.float32),
    )(sym_pad, W, b.reshape(1, D))


def _sc_gather_body(table_hbm, idx_hbm, out_hbm, idx_v, rows_v, *sems):
    gs, ws = sems[:NBUF], sems[NBUF:]
    wid = lax.axis_index("s") * NC + lax.axis_index("c")
    row0 = wid * N_CHUNKS
    pltpu.sync_copy(idx_hbm.at[pl.ds(row0, N_CHUNKS)], idx_v)

    def gather_start(ci, b):
        pltpu.make_async_copy(
            table_hbm.at[idx_v.at[ci]], rows_v.at[b], gs[b]
        ).start()

    def gather_wait(b):
        pltpu.make_async_copy(
            table_hbm.at[idx_v.at[0]], rows_v.at[b], gs[b]
        ).wait()

    def write_start(ci, b):
        pltpu.make_async_copy(
            rows_v.at[b], out_hbm.at[pl.ds((row0 + ci) * CH, CH)], ws[b]
        ).start()

    def write_wait(b):
        pltpu.make_async_copy(
            rows_v.at[b], out_hbm.at[pl.ds(0, CH)], ws[b]
        ).wait()

    for ci in range(NBUF - 1):
        gather_start(ci, ci)

    def step(i, carry):
        for b in range(NBUF):
            ci = i * NBUF + b
            gather_wait(b)
            write_start(ci, b)
            nci = ci + NBUF - 1
            bn = (b + NBUF - 1) % NBUF

            @pl.when(nci < N_CHUNKS)
            def _():
                @pl.when(ci >= 1)
                def _():
                    write_wait(bn)

                gather_start(nci, bn)

        return carry

    lax.fori_loop(0, N_CHUNKS // NBUF, step, 0)
    for b in range(NBUF):
        write_wait(b)


@functools.partial(jax.jit)
def kernel(QR, symbols, W, b):
    table = _make_table(symbols, W, b)
    idx = QR.reshape(NW * N_CHUNKS, CH).astype(jnp.int32)
    mesh = plsc.VectorSubcoreMesh(core_axis_name="c", subcore_axis_name="s")
    gather = pl.kernel(
        _sc_gather_body,
        out_type=jax.ShapeDtypeStruct((BT, D), jnp.float32),
        mesh=mesh,
        scratch_types=(
            [
                pltpu.VMEM((NW * N_CHUNKS // NW, CH), jnp.int32),
                pltpu.VMEM((NBUF, CH, D), jnp.float32),
            ]
            + [pltpu.SemaphoreType.DMA] * (2 * NBUF)
        ),
        compiler_params=pltpu.CompilerParams(use_tc_tiling_on_sc=False),
    )
    out = gather(table, idx)
    return out.reshape(B, L, D)
